# Initial kernel scaffold; baseline (speedup 1.0000x reference)
import jax
import jax.numpy as jnp
from jax.experimental import pallas as pl


def kernel(input_word, target, output_matrix):
    raise NotImplementedError("write your pallas kernel here")


# speedup vs baseline: 1.6487x; 1.6487x over previous
import functools

import jax
import jax.numpy as jnp
from jax import lax
from jax.experimental import pallas as pl
from jax.experimental.pallas import tpu as pltpu
from jax.experimental.pallas import tpu_sc as plsc

_VOCAB = 100000
_DIM = 128
_BATCH = 4096
_L = 17
_LP = 32
_C = 16
_NC = 2
_NS = 16
_NW = _NC * _NS
_RW = _BATCH // _NW
_NCH = _RW // _C


def _sc_body(input_hbm, target_hbm, table_hbm, out_hbm,
             tgt_v, idx_v, w_v, e_v, dots_v, sem):
    wid = lax.axis_index("s") * _NC + lax.axis_index("c")
    lanes = lax.iota(jnp.int32, 16)

    def chunk_body(ch, carry):
        base = wid * _RW + ch * _C
        pltpu.sync_copy(target_hbm.at[pl.ds(base, _C)], tgt_v)
        pltpu.sync_copy(input_hbm.at[pl.ds(base * _DIM, _C * _DIM)], w_v)

        m = tgt_v[...] + _VOCAB
        for kk in range(1, _L + 1):
            mk = jnp.right_shift(m, kk)
            idx_v[kk - 1] = jnp.where(mk >= 1, mk - 1, _VOCAB)

        copies = [
            pltpu.async_copy(table_hbm.at[idx_v.at[kk]],
                             e_v.at[pl.ds(kk * _C, _C)], sem)
            for kk in range(_L)
        ]
        for cp in copies:
            cp.wait()

        def row_body(b, carry2):
            wb = [w_v[pl.ds(b * _DIM + c * 16, 16)] for c in range(8)]
            dots0 = jnp.zeros((16,), jnp.float32)
            dots1 = jnp.zeros((16,), jnp.float32)
            for kk in range(_L):
                row = kk * _C + b
                acc = e_v[row, pl.ds(0, 16)] * wb[0]
                for c in range(1, 8):
                    acc = acc + e_v[row, pl.ds(c * 16, 16)] * wb[c]
                for s in (1, 2, 4, 8):
                    acc = acc + acc.at[lanes ^ s].get(
                        mode="promise_in_bounds")
                if kk < 16:
                    dots0 = jnp.where(lanes == kk, acc, dots0)
                else:
                    dots1 = jnp.where(lanes == 0, acc, dots1)
            dots_v[pl.ds(b * _LP, 16)] = dots0
            dots_v[pl.ds(b * _LP + 16, 16)] = dots1
            return carry2

        lax.fori_loop(0, _C, row_body, 0)
        pltpu.sync_copy(dots_v, out_hbm.at[pl.ds(base * _LP, _C * _LP)])
        return carry

    lax.fori_loop(0, _NCH, chunk_body, 0)


_sc_dots = functools.partial(
    pl.kernel,
    mesh=plsc.VectorSubcoreMesh(core_axis_name="c", subcore_axis_name="s"),
    out_type=jax.ShapeDtypeStruct((_BATCH * _LP,), jnp.float32),
    scratch_types=[
        pltpu.VMEM((_C,), jnp.int32),
        pltpu.VMEM((_L, _C), jnp.int32),
        pltpu.VMEM((_C * _DIM,), jnp.float32),
        pltpu.VMEM((_L * _C, _DIM), jnp.float32),
        pltpu.VMEM((_C * _LP,), jnp.float32),
        pltpu.SemaphoreType.DMA,
    ],
)(_sc_body)


def _tc_loss_body(dots_ref, tgt_ref, out_ref):
    m = tgt_ref[...] + _VOCAB
    col = lax.broadcasted_iota(jnp.int32, (_BATCH, _LP), 1)
    mk = jnp.right_shift(m, col + 1)
    turn = jnp.where((jnp.right_shift(m, col) & 1) == 0, 1.0, -1.0)
    coef = jnp.where(mk >= 1, turn, 0.0)
    x = dots_ref[...] * coef
    ls = jnp.where(col < _L, jax.nn.log_sigmoid(x), 0.0)
    out_ref[0, 0] = -jnp.sum(ls) / _BATCH


def _tc_loss(dots2d, tgt2d):
    return pl.pallas_call(
        _tc_loss_body,
        out_shape=jax.ShapeDtypeStruct((1, 1), jnp.float32),
        out_specs=pl.BlockSpec(memory_space=pltpu.SMEM),
    )(dots2d, tgt2d)


def kernel(input_word, target, output_matrix):
    dots_flat = _sc_dots(input_word.reshape(-1), target, output_matrix)
    loss = _tc_loss(dots_flat.reshape(_BATCH, _LP),
                    target.reshape(_BATCH, 1))
    return loss[0, 0]
